# baseline (speedup 1.0000x reference)
import functools

import jax
import jax.numpy as jnp
from jax import lax
from jax.experimental import pallas as pl
from jax.experimental.pallas import tpu as pltpu
from jax.experimental.pallas import tpu_sc as plsc

_VOCAB = 100000
_EMBED = 300
_N_TOKENS = 16384
_EPAD = 384

_NC = 2
_NS = 16
_NW = _NC * _NS
_TPW = _N_TOKENS // _NW
_CH = 128
_NCHUNK = _TPW // _CH
_NSLOT = _CH // 16


def _sc_gather_t(idx, table, tail):
    mesh = plsc.VectorSubcoreMesh(core_axis_name="c", subcore_axis_name="s")

    @functools.partial(
        pl.kernel,
        mesh=mesh,
        out_type=jax.ShapeDtypeStruct((_EMBED, _N_TOKENS), jnp.float32),
        scratch_types=[
            pltpu.VMEM((_CH,), jnp.int32),
            pltpu.VMEM((_CH, _EPAD), jnp.float32),
            pltpu.VMEM((_EMBED, _CH), jnp.float32),
            pltpu.SemaphoreType.DMA,
        ],
        compiler_params=pltpu.CompilerParams(
            use_tc_tiling_on_sc=True, needs_layout_passes=False),
    )
    def k(idx_hbm, table_hbm, tail_hbm, out_hbm, idx_v, g_v, t_v, sem):
        wid = lax.axis_index("s") * _NC + lax.axis_index("c")
        base = wid * _TPW
        lane = lax.iota(jnp.int32, 16)
        for j in range(_NCHUNK):
            off = base + j * _CH
            pltpu.sync_copy(idx_hbm.at[pl.ds(off, _CH)], idx_v)
            cps = []
            for p in range(3):
                if p < 2:
                    src = table_hbm.at[idx_v, pl.ds(p * 128, 128)]
                else:
                    src = tail_hbm.at[idx_v]
                cps.append(
                    pltpu.async_copy(src, g_v.at[:, pl.ds(p * 128, 128)], sem))
            for cp in cps:
                cp.wait()

            def transpose_row(e, _):
                col = jnp.full((16,), e, jnp.int32)
                for s in range(_NSLOT):
                    vals = plsc.load_gather(g_v, [lane + s * 16, col])
                    t_v[e, pl.ds(s * 16, 16)] = vals
                return _

            lax.fori_loop(0, _EMBED, transpose_row, 0)
            pltpu.sync_copy(t_v, out_hbm.at[:, pl.ds(off, _CH)])

    return k(idx, table, tail)


def kernel(input, table):
    idx = input.astype(jnp.int32)
    tail = jnp.pad(lax.slice_in_dim(table, 2 * 128, _EMBED, axis=1),
                   ((0, 0), (0, _EPAD - _EMBED)))
    return _sc_gather_t(idx, table, tail)


# speedup vs baseline: 2.1858x; 2.1858x over previous
import functools

import jax
import jax.numpy as jnp
from jax import lax
from jax.experimental import pallas as pl
from jax.experimental.pallas import tpu as pltpu
from jax.experimental.pallas import tpu_sc as plsc

_VOCAB = 100000
_EMBED = 300
_N_TOKENS = 16384
_EPAD = 384

_NC = 2
_NS = 16
_NW = _NC * _NS
_TPW = _N_TOKENS // _NW
_CH = 128
_NCHUNK = _TPW // _CH

_VB = 2048


def _tc_transpose_pad(table_t):
    def body(x_ref, o_ref):
        o_ref[...] = jnp.pad(x_ref[...].T, ((0, 0), (0, _EPAD - _EMBED)))

    grid = (_VOCAB + _VB - 1) // _VB
    return pl.pallas_call(
        body,
        grid=(grid,),
        in_specs=[pl.BlockSpec((_EMBED, _VB), lambda i: (0, i))],
        out_specs=pl.BlockSpec((_VB, _EPAD), lambda i: (i, 0)),
        out_shape=jax.ShapeDtypeStruct((_VOCAB, _EPAD), jnp.float32),
    )(table_t)


def _sc_gather(idx, table_pad):
    mesh = plsc.VectorSubcoreMesh(core_axis_name="c", subcore_axis_name="s")

    @functools.partial(
        pl.kernel,
        mesh=mesh,
        out_type=jax.ShapeDtypeStruct((_N_TOKENS, _EPAD), jnp.float32),
        scratch_types=[
            pltpu.VMEM((_CH,), jnp.int32),
            pltpu.VMEM((_CH, _EPAD), jnp.float32),
            pltpu.SemaphoreType.DMA,
        ],
        compiler_params=pltpu.CompilerParams(use_tc_tiling_on_sc=True),
    )
    def k(idx_hbm, table_hbm, out_hbm, idx_v, rows_v, sem):
        wid = lax.axis_index("s") * _NC + lax.axis_index("c")
        base = wid * _TPW
        for j in range(_NCHUNK):
            off = base + j * _CH
            pltpu.sync_copy(idx_hbm.at[pl.ds(off, _CH)], idx_v)
            pltpu.async_copy(table_hbm.at[idx_v], rows_v, sem).wait()
            pltpu.sync_copy(rows_v, out_hbm.at[pl.ds(off, _CH)])

    return k(idx, table_pad)


_TB = 1024


def _tc_transpose(x):
    def body(x_ref, o_ref):
        o_ref[...] = x_ref[:, :_EMBED].T

    return pl.pallas_call(
        body,
        grid=(_N_TOKENS // _TB,),
        in_specs=[pl.BlockSpec((_TB, _EPAD), lambda i: (i, 0))],
        out_specs=pl.BlockSpec((_EMBED, _TB), lambda i: (0, i)),
        out_shape=jax.ShapeDtypeStruct((_EMBED, _N_TOKENS), jnp.float32),
    )(x)


def kernel(input, table):
    idx = input.astype(jnp.int32)
    table_pad = _tc_transpose_pad(table.T)
    gathered = _sc_gather(idx, table_pad)
    return _tc_transpose(gathered)
